# Initial kernel scaffold; baseline (speedup 1.0000x reference)
import jax
import jax.numpy as jnp
from jax.experimental import pallas as pl


def kernel(x_0, x_0_batch, num_cells_0, W1, b1, g1, bt1, W2, b2, g2, bt2, W3, b3):
    raise NotImplementedError("write your pallas kernel here")


# speedup vs baseline: 4.1571x; 4.1571x over previous
import functools
import math

import jax
import jax.numpy as jnp
from jax import lax
from jax.experimental import pallas as pl
from jax.experimental.pallas import tpu as pltpu
from jax.experimental.pallas import tpu_sc as plsc

N = 100000
D = 128
S = 1024

NC = 2
NS = 16
NW = NC * NS

BLK = 128
NFULL = N // BLK
TAIL = N - NFULL * BLK
TBASE = NFULL * BLK
MAXB = (NFULL + NW - 1) // NW
ROWS_PER_TILE = S // NS


@functools.partial(
    pl.kernel,
    mesh=plsc.VectorSubcoreMesh(core_axis_name="c", subcore_axis_name="s"),
    out_type=jax.ShapeDtypeStruct((2 * S, D), jnp.float32),
    scratch_types=[
        pltpu.VMEM((BLK, D), jnp.float32),
        pltpu.VMEM((BLK,), jnp.int32),
        pltpu.VMEM((TAIL,), jnp.int32),
        pltpu.VMEM_SHARED((S, D), jnp.float32),
    ],
)
def _sc_segment_sum(x_hbm, idx_hbm, z_hbm, out_hbm, xbuf, ibuf, ibuf_t, acc):
    cid = lax.axis_index("c")
    sid = lax.axis_index("s")
    wid = sid * NC + cid

    pltpu.sync_copy(z_hbm, acc.at[pl.ds(sid * ROWS_PER_TILE, ROWS_PER_TILE)])
    plsc.subcore_barrier()

    def body(i, carry):
        bi = wid + i * NW

        @pl.when(bi < NFULL)
        def _():
            base = bi * BLK
            pltpu.sync_copy(x_hbm.at[pl.ds(base, BLK)], xbuf)
            pltpu.sync_copy(idx_hbm.at[pl.ds(base, BLK)], ibuf)
            pltpu.sync_copy(xbuf, acc.at[ibuf], add=True)

        return carry

    lax.fori_loop(0, MAXB, body, 0)

    @pl.when(wid == NW - 1)
    def _():
        pltpu.sync_copy(x_hbm.at[pl.ds(TBASE, TAIL)], xbuf.at[pl.ds(0, TAIL)])
        pltpu.sync_copy(idx_hbm.at[pl.ds(TBASE, TAIL)], ibuf_t)
        pltpu.sync_copy(xbuf.at[pl.ds(0, TAIL)], acc.at[ibuf_t], add=True)

    plsc.subcore_barrier()

    row0 = sid * ROWS_PER_TILE
    pltpu.sync_copy(
        acc.at[pl.ds(row0, ROWS_PER_TILE)],
        out_hbm.at[pl.ds(cid * S + row0, ROWS_PER_TILE)],
    )


_INV_BN = 1.0 / math.sqrt(1.0 + 1e-5)


def _mlp_body(p_ref, w1, b1, g1, t1, w2, b2, g2, t2, w3, b3, o_ref):
    pooled = p_ref[0] + p_ref[1]
    h = jnp.dot(pooled, w1[...], preferred_element_type=jnp.float32) + b1[...]
    h = g1[...] * (h * _INV_BN) + t1[...]
    h = jnp.maximum(h, 0.0)
    h = jnp.dot(h, w2[...], preferred_element_type=jnp.float32) + b2[...]
    h = g2[...] * (h * _INV_BN) + t2[...]
    h = jnp.maximum(h, 0.0)
    o_ref[...] = jnp.dot(h, w3[...], preferred_element_type=jnp.float32) + b3[...]


def _mlp(partial, W1, b1, g1, bt1, W2, b2, g2, bt2, W3, b3):
    return pl.pallas_call(
        _mlp_body,
        out_shape=jax.ShapeDtypeStruct((S, 1), jnp.float32),
    )(partial, W1, b1, g1, bt1, W2, b2, g2, bt2, W3, b3)


def kernel(x_0, x_0_batch, num_cells_0, W1, b1, g1, bt1, W2, b2, g2, bt2, W3, b3):
    idx = jnp.squeeze(x_0_batch).astype(jnp.int32)
    zrows = jnp.zeros((ROWS_PER_TILE, D), jnp.float32)
    partial = _sc_segment_sum(x_0, idx, zrows)
    partial = partial.reshape(2, S, D)
    return _mlp(
        partial,
        W1, b1.reshape(1, D), g1.reshape(1, D), bt1.reshape(1, D),
        W2, b2.reshape(1, 2 * D), g2.reshape(1, 2 * D), bt2.reshape(1, 2 * D),
        W3, b3.reshape(1, 1),
    )


# speedup vs baseline: 6.2197x; 1.4962x over previous
import functools
import math

import jax
import jax.numpy as jnp
from jax import lax
from jax.experimental import pallas as pl
from jax.experimental.pallas import tpu as pltpu
from jax.experimental.pallas import tpu_sc as plsc

N = 100000
D = 128
S = 1024

NC = 2
NS = 16
NW = NC * NS

CH = 128
BLK = 256
NBF = N // BLK
TAIL = N - NBF * BLK
TBASE = NBF * BLK
NPAD = ((N + CH - 1) // CH) * CH
MAXB = (NBF + NW - 1) // NW
ROWS_PER_TILE = S // NS


@functools.partial(
    pl.kernel,
    mesh=plsc.VectorSubcoreMesh(core_axis_name="c", subcore_axis_name="s"),
    out_type=jax.ShapeDtypeStruct((2 * S, D), jnp.float32),
    scratch_types=[
        pltpu.VMEM((BLK, D), jnp.float32),
        pltpu.VMEM((BLK, D), jnp.float32),
        pltpu.VMEM((BLK // CH, CH), jnp.int32),
        pltpu.VMEM((BLK // CH, CH), jnp.int32),
        pltpu.VMEM((TAIL - CH,), jnp.int32),
        pltpu.VMEM_SHARED((S, D), jnp.float32),
        pltpu.SemaphoreType.DMA,
        pltpu.SemaphoreType.DMA,
    ],
)
def _sc_segment_sum(x_hbm, idx2_hbm, idxf_hbm, z_hbm, out_hbm,
                    xbuf0, xbuf1, ibuf0, ibuf1, ibuf_t, acc, sem0, sem1):
    cid = lax.axis_index("c")
    sid = lax.axis_index("s")
    wid = sid * NC + cid

    xbufs = (xbuf0, xbuf1)
    ibufs = (ibuf0, ibuf1)
    sems = (sem0, sem1)

    pltpu.sync_copy(z_hbm, acc.at[pl.ds(sid * ROWS_PER_TILE, ROWS_PER_TILE)])
    plsc.subcore_barrier()

    def copies_for(i):
        p = i % 2
        bi = wid + i * NW
        base = bi * BLK
        return bi, [
            (x_hbm.at[pl.ds(base, BLK)], xbufs[p], sems[p]),
            (idx2_hbm.at[pl.ds(bi * (BLK // CH), BLK // CH)], ibufs[p], sems[p]),
        ]

    def issue(i):
        bi, copies = copies_for(i)

        @pl.when(bi < NBF)
        def _():
            for src, dst, sem in copies:
                pltpu.async_copy(src, dst, sem)

    def wait_and_scatter(i):
        p = i % 2
        bi, copies = copies_for(i)

        @pl.when(bi < NBF)
        def _():
            for src, dst, sem in copies:
                pltpu.make_async_copy(src, dst, sem).wait()
            for j in range(BLK // CH):
                pltpu.sync_copy(
                    xbufs[p].at[pl.ds(j * CH, CH)],
                    acc.at[ibufs[p].at[j]],
                    add=True,
                )

    issue(0)
    for i in range(MAXB):
        if i + 1 < MAXB:
            issue(i + 1)
        wait_and_scatter(i)

    @pl.when(wid == NW - 1)
    def _():
        pltpu.sync_copy(x_hbm.at[pl.ds(TBASE, TAIL)], xbuf0.at[pl.ds(0, TAIL)])
        pltpu.sync_copy(idx2_hbm.at[pl.ds(TBASE // CH, 1)], ibuf0.at[pl.ds(0, 1)])
        pltpu.sync_copy(idxf_hbm.at[pl.ds(TBASE + CH, TAIL - CH)], ibuf_t)
        pltpu.sync_copy(xbuf0.at[pl.ds(0, CH)], acc.at[ibuf0.at[0]], add=True)
        pltpu.sync_copy(xbuf0.at[pl.ds(CH, TAIL - CH)], acc.at[ibuf_t], add=True)

    plsc.subcore_barrier()

    row0 = sid * ROWS_PER_TILE
    pltpu.sync_copy(
        acc.at[pl.ds(row0, ROWS_PER_TILE)],
        out_hbm.at[pl.ds(cid * S + row0, ROWS_PER_TILE)],
    )


_INV_BN = 1.0 / math.sqrt(1.0 + 1e-5)


def _mlp_body(p_ref, w1, b1, g1, t1, w2, b2, g2, t2, w3, b3, o_ref):
    pooled = p_ref[0] + p_ref[1]
    h = jnp.dot(pooled, w1[...], preferred_element_type=jnp.float32) + b1[...]
    h = g1[...] * (h * _INV_BN) + t1[...]
    h = jnp.maximum(h, 0.0)
    h = jnp.dot(h, w2[...], preferred_element_type=jnp.float32) + b2[...]
    h = g2[...] * (h * _INV_BN) + t2[...]
    h = jnp.maximum(h, 0.0)
    o_ref[...] = jnp.dot(h, w3[...], preferred_element_type=jnp.float32) + b3[...]


def _mlp(partial, W1, b1, g1, bt1, W2, b2, g2, bt2, W3, b3):
    return pl.pallas_call(
        _mlp_body,
        out_shape=jax.ShapeDtypeStruct((S, 1), jnp.float32),
    )(partial, W1, b1, g1, bt1, W2, b2, g2, bt2, W3, b3)


def kernel(x_0, x_0_batch, num_cells_0, W1, b1, g1, bt1, W2, b2, g2, bt2, W3, b3):
    idx = jnp.squeeze(x_0_batch).astype(jnp.int32)
    idx_pad = jnp.concatenate([idx, jnp.zeros((NPAD - N,), jnp.int32)])
    idx2 = idx_pad.reshape(NPAD // CH, CH)
    zrows = jnp.zeros((ROWS_PER_TILE, D), jnp.float32)
    partial = _sc_segment_sum(x_0, idx2, idx_pad, zrows)
    partial = partial.reshape(2, S, D)
    return _mlp(
        partial,
        W1, b1.reshape(1, D), g1.reshape(1, D), bt1.reshape(1, D),
        W2, b2.reshape(1, 2 * D), g2.reshape(1, 2 * D), bt2.reshape(1, 2 * D),
        W3, b3.reshape(1, 1),
    )
